# baseline (speedup 1.0000x reference)
import functools

import jax
import jax.numpy as jnp
from jax import lax
from jax.experimental import pallas as pl
from jax.experimental.pallas import tpu as pltpu
from jax.experimental.pallas import tpu_sc as plsc

_N = 10000
_D = 128
_DG = 32
_NG = 4
_E = 320000
_ET = _E + _N
_NP = 10240
_NWS = 16
_C = 128
_NCH = 164
_NB = 4
_PF = 2
_EW = _NCH * _C
_EP = _NWS * _EW
_PAD_DST = 10200
_RPT = _NP // 16
_NEG_SLOPE = 0.2


_BLK = 512
_GRID = _NP // _BLK


def _proj_body(x_ref, w_ref, a_ref, h_ref, s_ref):
    h = jnp.dot(x_ref[...], w_ref[...], preferred_element_type=jnp.float32)
    for g in range(_NG):
        h_ref[g] = h[:, g * _DG:(g + 1) * _DG]
    s_ref[...] = jnp.dot(h, a_ref[...], preferred_element_type=jnp.float32)


_proj = pl.pallas_call(
    _proj_body,
    grid=(_GRID,),
    in_specs=[
        pl.BlockSpec((_BLK, _D), lambda i: (i, 0)),
        pl.BlockSpec((_D, _D), lambda i: (0, 0)),
        pl.BlockSpec((_D, _D), lambda i: (0, 0)),
    ],
    out_specs=[
        pl.BlockSpec((_NG, _BLK, _DG), lambda i: (0, i, 0)),
        pl.BlockSpec((_BLK, _D), lambda i: (i, 0)),
    ],
    out_shape=[
        jax.ShapeDtypeStruct((_NG, _NP, _DG), jnp.float32),
        jax.ShapeDtypeStruct((_NP, _D), jnp.float32),
    ],
)


def _mid_body(num_ref, den_ref, b_ref, w_ref, a_ref, h_ref, s_ref):
    num = jnp.concatenate([num_ref[g] for g in range(_NG)], axis=1)
    den = den_ref[...]
    out = num / (den + 1e-16) + b_ref[...]
    out = jnp.maximum(out, 0.0)
    h = jnp.dot(out, w_ref[...], preferred_element_type=jnp.float32)
    for g in range(_NG):
        h_ref[g] = h[:, g * _DG:(g + 1) * _DG]
    s_ref[...] = jnp.dot(h, a_ref[...], preferred_element_type=jnp.float32)


_mid = pl.pallas_call(
    _mid_body,
    grid=(_GRID,),
    in_specs=[
        pl.BlockSpec((_NG, _BLK, _DG), lambda i: (0, i, 0)),
        pl.BlockSpec((_BLK, 1), lambda i: (i, 0)),
        pl.BlockSpec((1, _D), lambda i: (0, 0)),
        pl.BlockSpec((_D, _D), lambda i: (0, 0)),
        pl.BlockSpec((_D, _D), lambda i: (0, 0)),
    ],
    out_specs=[
        pl.BlockSpec((_NG, _BLK, _DG), lambda i: (0, i, 0)),
        pl.BlockSpec((_BLK, _D), lambda i: (i, 0)),
    ],
    out_shape=[
        jax.ShapeDtypeStruct((_NG, _NP, _DG), jnp.float32),
        jax.ShapeDtypeStruct((_NP, _D), jnp.float32),
    ],
)


def _fin_body(num_ref, den_ref, b_ref, o_ref):
    num = jnp.concatenate([num_ref[g] for g in range(_NG)], axis=1)
    den = den_ref[...]
    out = num / (den + 1e-16) + b_ref[...]
    o_ref[...] = jnp.maximum(out, 0.0)


_fin = pl.pallas_call(
    _fin_body,
    grid=(_GRID,),
    in_specs=[
        pl.BlockSpec((_NG, _BLK, _DG), lambda i: (0, i, 0)),
        pl.BlockSpec((_BLK, 1), lambda i: (i, 0)),
        pl.BlockSpec((1, _D), lambda i: (0, 0)),
    ],
    out_specs=pl.BlockSpec((_BLK, _D), lambda i: (i, 0)),
    out_shape=jax.ShapeDtypeStruct((_NP, _D), jnp.float32),
)


_mesh = plsc.VectorSubcoreMesh(core_axis_name="c", subcore_axis_name="s")


@functools.partial(
    pl.kernel,
    mesh=_mesh,
    compiler_params=pltpu.CompilerParams(
        needs_layout_passes=False, use_tc_tiling_on_sc=False),
    out_type=[
        jax.ShapeDtypeStruct((_NG, _NP, _DG), jnp.float32),
        jax.ShapeDtypeStruct((1, _NP), jnp.float32),
    ],
    scratch_types=[
        pltpu.VMEM((_NCH, _C), jnp.int32),
        pltpu.VMEM((_NCH, _C), jnp.int32),
        pltpu.VMEM((_NP,), jnp.float32),
        pltpu.VMEM((_NP,), jnp.float32),
        pltpu.VMEM((_NCH, _C), jnp.float32),
    ] + [pltpu.VMEM((_C, _DG), jnp.float32) for _ in range(_NB)] + [
        pltpu.VMEM((640,), jnp.float32),
        pltpu.VMEM_SHARED((_NP, _DG), jnp.float32),
        pltpu.VMEM_SHARED((_NP,), jnp.float32),
    ] + [pltpu.SemaphoreType.DMA for _ in range(2 * _NB + 1)],
)
def _edge_pass(h_hbm, as_hbm, ad_hbm, src_hbm, dst_hbm, num_out, den_out,
               src_v, dst_v, as_v, ad_v, t_v, *rest):
    rows = rest[:_NB]
    zden_v = rest[_NB]
    num_sh = rest[_NB + 1]
    den_sh = rest[_NB + 2]
    gsems = rest[_NB + 3:2 * _NB + 3]
    ssems = rest[2 * _NB + 3:3 * _NB + 3]
    dsem = rest[3 * _NB + 3]
    cid = lax.axis_index("c")
    sid = lax.axis_index("s")

    pltpu.sync_copy(src_hbm.at[sid], src_v)
    pltpu.sync_copy(dst_hbm.at[sid], dst_v)
    pltpu.sync_copy(as_hbm, as_v)
    pltpu.sync_copy(ad_hbm, ad_v)

    zeros16 = jnp.zeros((16,), jnp.float32)

    def _zden(i, _):
        zden_v[pl.ds(i * 16, 16)] = zeros16
        return 0

    lax.fori_loop(0, 640 // 16, _zden, 0)

    def _tloop(j, _):
        for g in range(8):
            sl = pl.ds(g * 16, 16)
            si = src_v[j, sl]
            di = dst_v[j, sl]
            e = plsc.load_gather(as_v, [si]) + plsc.load_gather(ad_v, [di])
            e = jnp.where(e > 0, e, _NEG_SLOPE * e)
            t_v[j, sl] = jnp.exp(e)
        return 0

    lax.fori_loop(0, _NCH, _tloop, 0)

    rbase = sid * _RPT

    for half in range(2):
        grp = cid * 2 + half

        def _zrows(i, _):
            r = i // (_DG // 16)
            g = i % (_DG // 16)
            rows[0][r, pl.ds(g * 16, 16)] = zeros16
            return 0

        lax.fori_loop(0, _C * (_DG // 16), _zrows, 0)
        for k in range(_RPT // _C):
            pltpu.sync_copy(rows[0], num_sh.at[pl.ds(rbase + k * _C, _C)])
        if _RPT % _C:
            pltpu.sync_copy(
                rows[0].at[pl.ds(0, _RPT % _C)],
                num_sh.at[pl.ds(rbase + (_RPT // _C) * _C, _RPT % _C)])
        if half == 0:
            pltpu.sync_copy(zden_v.at[pl.ds(0, _RPT)],
                            den_sh.at[pl.ds(rbase, _RPT)])

        plsc.subcore_barrier()

        h_grp = h_hbm.at[grp]

        for b in range(_PF):
            pltpu.async_copy(h_grp.at[src_v.at[b]], rows[b], gsems[b])

        def _outer(io, _):
            for k in range(_NB):
                j = io * _NB + k
                pltpu.make_async_copy(
                    h_grp.at[src_v.at[j]], rows[k], gsems[k]).wait()
                if half == 0:
                    dcopy = pltpu.async_copy(
                        t_v.at[j], den_sh.at[dst_v.at[j]], dsem, add=True)

                def _scale(q, _c, k=k, j=j):
                    t16 = t_v[j, pl.ds(q * 16, 16)]
                    for l in range(16):
                        ts = t16[l]
                        r = q * 16 + l
                        for g in range(_DG // 16):
                            sl = pl.ds(g * 16, 16)
                            rows[k][r, sl] = rows[k][r, sl] * ts
                    return 0

                lax.fori_loop(0, _C // 16, _scale, 0)
                pltpu.async_copy(
                    rows[k], num_sh.at[dst_v.at[j]], ssems[k], add=True)
                if half == 0:
                    dcopy.wait()

                kp = (k - _PF) % _NB
                jp = j - _PF

                @pl.when(jp >= 0)
                def _(kp=kp, jp=jp):
                    pltpu.make_async_copy(
                        rows[kp], num_sh.at[dst_v.at[jp]], ssems[kp]).wait()

                @pl.when(j + _PF < _NCH)
                def _(kp=kp, j=j):
                    pltpu.async_copy(
                        h_grp.at[src_v.at[j + _PF]], rows[kp], gsems[kp])
            return 0

        lax.fori_loop(0, _NCH // _NB, _outer, 0)

        for i in range(_PF):
            j = _NCH - _PF + i
            k = j % _NB
            pltpu.make_async_copy(
                rows[k], num_sh.at[dst_v.at[j]], ssems[k]).wait()

        plsc.subcore_barrier()

        pltpu.sync_copy(num_sh.at[pl.ds(rbase, _RPT)],
                        num_out.at[grp, pl.ds(rbase, _RPT)])
        if half == 0:
            @pl.when(cid == 0)
            def _():
                pltpu.sync_copy(den_sh.at[pl.ds(rbase, _RPT)],
                                den_out.at[0, pl.ds(rbase, _RPT)])

        plsc.subcore_barrier()


def _acat(a_s, a_d):
    z = jnp.zeros((_D, _D - 2), jnp.float32)
    return jnp.concatenate([a_s[:, None], a_d[:, None], z], axis=1)


def kernel(x, edge_index, W1, a_src1, a_dst1, b1, W2, a_src2, a_dst2, b2):
    xp = jnp.pad(x, ((0, _NP - _N), (0, 0)))

    ar = jnp.arange(_N, dtype=edge_index.dtype)
    src = jnp.concatenate([edge_index[0], ar])
    dst = jnp.concatenate([edge_index[1], ar])
    src = jnp.pad(src, (0, _EP - _ET))
    dst = jnp.pad(dst, (0, _EP - _ET), constant_values=_PAD_DST)
    srcr = src.reshape(_NWS, _NCH, _C)
    dstr = dst.reshape(_NWS, _NCH, _C)

    h1, s1 = _proj(xp, W1, _acat(a_src1, a_dst1))
    num1, den1 = _edge_pass(h1, s1[:, 0], s1[:, 1], srcr, dstr)
    h2, s2 = _mid(num1, den1.reshape(_NP, 1), b1.reshape(1, _D), W2,
                  _acat(a_src2, a_dst2))
    num2, den2 = _edge_pass(h2, s2[:, 0], s2[:, 1], srcr, dstr)
    out = _fin(num2, den2.reshape(_NP, 1), b2.reshape(1, _D))
    return out[:_N]


# speedup vs baseline: 1.3661x; 1.3661x over previous
import functools

import jax
import jax.numpy as jnp
from jax import lax
from jax.experimental import pallas as pl
from jax.experimental.pallas import tpu as pltpu
from jax.experimental.pallas import tpu_sc as plsc

_N = 10000
_D = 128
_DG = 32
_NG = 4
_E = 320000
_ET = _E + _N
_NP = 10240
_NWS = 16
_C = 128
_NCH = 162
_NB = 6
_PF = 2
_EW = _NCH * _C
_EP = _NWS * _EW
_PAD_DST = 10200
_RPT = _NP // 16
_NEG_SLOPE = 0.2


_BLK = 512
_GRID = _NP // _BLK


def _proj_body(x_ref, w_ref, a_ref, h_ref, s_ref):
    h = jnp.dot(x_ref[...], w_ref[...], preferred_element_type=jnp.float32)
    for g in range(_NG):
        h_ref[g] = h[:, g * _DG:(g + 1) * _DG]
    s_ref[...] = jnp.dot(h, a_ref[...], preferred_element_type=jnp.float32)


_proj = pl.pallas_call(
    _proj_body,
    grid=(_GRID,),
    in_specs=[
        pl.BlockSpec((_BLK, _D), lambda i: (i, 0)),
        pl.BlockSpec((_D, _D), lambda i: (0, 0)),
        pl.BlockSpec((_D, _D), lambda i: (0, 0)),
    ],
    out_specs=[
        pl.BlockSpec((_NG, _BLK, _DG), lambda i: (0, i, 0)),
        pl.BlockSpec((_BLK, _D), lambda i: (i, 0)),
    ],
    out_shape=[
        jax.ShapeDtypeStruct((_NG, _NP, _DG), jnp.float32),
        jax.ShapeDtypeStruct((_NP, _D), jnp.float32),
    ],
)


def _mid_body(num_ref, den_ref, b_ref, w_ref, a_ref, h_ref, s_ref):
    num = jnp.concatenate([num_ref[g] for g in range(_NG)], axis=1)
    den = den_ref[...]
    out = num / (den + 1e-16) + b_ref[...]
    out = jnp.maximum(out, 0.0)
    h = jnp.dot(out, w_ref[...], preferred_element_type=jnp.float32)
    for g in range(_NG):
        h_ref[g] = h[:, g * _DG:(g + 1) * _DG]
    s_ref[...] = jnp.dot(h, a_ref[...], preferred_element_type=jnp.float32)


_mid = pl.pallas_call(
    _mid_body,
    grid=(_GRID,),
    in_specs=[
        pl.BlockSpec((_NG, _BLK, _DG), lambda i: (0, i, 0)),
        pl.BlockSpec((_BLK, 1), lambda i: (i, 0)),
        pl.BlockSpec((1, _D), lambda i: (0, 0)),
        pl.BlockSpec((_D, _D), lambda i: (0, 0)),
        pl.BlockSpec((_D, _D), lambda i: (0, 0)),
    ],
    out_specs=[
        pl.BlockSpec((_NG, _BLK, _DG), lambda i: (0, i, 0)),
        pl.BlockSpec((_BLK, _D), lambda i: (i, 0)),
    ],
    out_shape=[
        jax.ShapeDtypeStruct((_NG, _NP, _DG), jnp.float32),
        jax.ShapeDtypeStruct((_NP, _D), jnp.float32),
    ],
)


def _fin_body(num_ref, den_ref, b_ref, o_ref):
    num = jnp.concatenate([num_ref[g] for g in range(_NG)], axis=1)
    den = den_ref[...]
    out = num / (den + 1e-16) + b_ref[...]
    o_ref[...] = jnp.maximum(out, 0.0)


_fin = pl.pallas_call(
    _fin_body,
    grid=(_GRID,),
    in_specs=[
        pl.BlockSpec((_NG, _BLK, _DG), lambda i: (0, i, 0)),
        pl.BlockSpec((_BLK, 1), lambda i: (i, 0)),
        pl.BlockSpec((1, _D), lambda i: (0, 0)),
    ],
    out_specs=pl.BlockSpec((_BLK, _D), lambda i: (i, 0)),
    out_shape=jax.ShapeDtypeStruct((_NP, _D), jnp.float32),
)


_mesh = plsc.VectorSubcoreMesh(core_axis_name="c", subcore_axis_name="s")


@functools.partial(
    pl.kernel,
    mesh=_mesh,
    compiler_params=pltpu.CompilerParams(
        needs_layout_passes=False, use_tc_tiling_on_sc=False),
    out_type=[
        jax.ShapeDtypeStruct((_NG, _NP, _DG), jnp.float32),
        jax.ShapeDtypeStruct((1, _NP), jnp.float32),
    ],
    scratch_types=[
        pltpu.VMEM((_NCH, _C), jnp.int32),
        pltpu.VMEM((_NCH, _C), jnp.int32),
        pltpu.VMEM((_NP,), jnp.float32),
        pltpu.VMEM((_NP,), jnp.float32),
        pltpu.VMEM((_NCH, _C), jnp.float32),
    ] + [pltpu.VMEM((_C, _DG), jnp.float32) for _ in range(_NB)] + [
        pltpu.VMEM((640,), jnp.float32),
        pltpu.VMEM_SHARED((_NP, _DG), jnp.float32),
        pltpu.VMEM_SHARED((_NP,), jnp.float32),
    ] + [pltpu.SemaphoreType.DMA for _ in range(_NB + 1)],
)
def _edge_pass(h_hbm, as_hbm, ad_hbm, src_hbm, dst_hbm, num_out, den_out,
               src_v, dst_v, as_v, ad_v, t_v, *rest):
    rows = rest[:_NB]
    zden_v = rest[_NB]
    num_sh = rest[_NB + 1]
    den_sh = rest[_NB + 2]
    gsems = rest[_NB + 3:2 * _NB + 3]
    dsem = rest[2 * _NB + 3]
    cid = lax.axis_index("c")
    sid = lax.axis_index("s")

    pltpu.sync_copy(src_hbm.at[sid], src_v)
    pltpu.sync_copy(dst_hbm.at[sid], dst_v)
    pltpu.sync_copy(as_hbm, as_v)
    pltpu.sync_copy(ad_hbm, ad_v)

    zeros16 = jnp.zeros((16,), jnp.float32)

    def _zden(i, _):
        zden_v[pl.ds(i * 16, 16)] = zeros16
        return 0

    lax.fori_loop(0, 640 // 16, _zden, 0)

    def _tloop(j, _):
        for g in range(8):
            sl = pl.ds(g * 16, 16)
            si = src_v[j, sl]
            di = dst_v[j, sl]
            e = plsc.load_gather(as_v, [si]) + plsc.load_gather(ad_v, [di])
            e = jnp.where(e > 0, e, _NEG_SLOPE * e)
            t_v[j, sl] = jnp.exp(e)
        return 0

    lax.fori_loop(0, _NCH, _tloop, 0)

    rbase = sid * _RPT

    for half in range(2):
        grp = cid * 2 + half

        def _zrows(i, _):
            r = i // (_DG // 16)
            g = i % (_DG // 16)
            rows[0][r, pl.ds(g * 16, 16)] = zeros16
            return 0

        lax.fori_loop(0, _C * (_DG // 16), _zrows, 0)
        for k in range(_RPT // _C):
            pltpu.sync_copy(rows[0], num_sh.at[pl.ds(rbase + k * _C, _C)])
        if _RPT % _C:
            pltpu.sync_copy(
                rows[0].at[pl.ds(0, _RPT % _C)],
                num_sh.at[pl.ds(rbase + (_RPT // _C) * _C, _RPT % _C)])
        if half == 0:
            pltpu.sync_copy(zden_v.at[pl.ds(0, _RPT)],
                            den_sh.at[pl.ds(rbase, _RPT)])

        plsc.subcore_barrier()

        h_grp = h_hbm.at[grp]

        for b in range(_NB):
            pltpu.async_copy(h_grp.at[src_v.at[b]], rows[b], gsems[b])

        def _outer(io, _):
            for k in range(_NB):
                j = io * _NB + k
                pltpu.make_async_copy(
                    h_grp.at[src_v.at[j]], rows[k], gsems[k]).wait()
                if half == 0:
                    dcopy = pltpu.async_copy(
                        t_v.at[j], den_sh.at[dst_v.at[j]], dsem, add=True)

                def _scale(q, _c, k=k, j=j):
                    t16 = t_v[j, pl.ds(q * 16, 16)]
                    for l in range(16):
                        ts = t16[l]
                        r = q * 16 + l
                        for g in range(_DG // 16):
                            sl = pl.ds(g * 16, 16)
                            rows[k][r, sl] = rows[k][r, sl] * ts
                    return 0

                lax.fori_loop(0, _C // 16, _scale, 0)
                pltpu.sync_copy(rows[k], num_sh.at[dst_v.at[j]], add=True)
                if half == 0:
                    dcopy.wait()

                @pl.when(j + _NB < _NCH)
                def _(k=k, j=j):
                    pltpu.async_copy(
                        h_grp.at[src_v.at[j + _NB]], rows[k], gsems[k])
            return 0

        lax.fori_loop(0, _NCH // _NB, _outer, 0)

        plsc.subcore_barrier()

        pltpu.sync_copy(num_sh.at[pl.ds(rbase, _RPT)],
                        num_out.at[grp, pl.ds(rbase, _RPT)])
        if half == 0:
            @pl.when(cid == 0)
            def _():
                pltpu.sync_copy(den_sh.at[pl.ds(rbase, _RPT)],
                                den_out.at[0, pl.ds(rbase, _RPT)])

        plsc.subcore_barrier()


def _acat(a_s, a_d):
    z = jnp.zeros((_D, _D - 2), jnp.float32)
    return jnp.concatenate([a_s[:, None], a_d[:, None], z], axis=1)


def kernel(x, edge_index, W1, a_src1, a_dst1, b1, W2, a_src2, a_dst2, b2):
    xp = jnp.pad(x, ((0, _NP - _N), (0, 0)))

    ar = jnp.arange(_N, dtype=edge_index.dtype)
    src = jnp.concatenate([edge_index[0], ar])
    dst = jnp.concatenate([edge_index[1], ar])
    src = jnp.pad(src, (0, _EP - _ET))
    dst = jnp.pad(dst, (0, _EP - _ET), constant_values=_PAD_DST)
    srcr = src.reshape(_NWS, _NCH, _C)
    dstr = dst.reshape(_NWS, _NCH, _C)

    h1, s1 = _proj(xp, W1, _acat(a_src1, a_dst1))
    num1, den1 = _edge_pass(h1, s1[:, 0], s1[:, 1], srcr, dstr)
    h2, s2 = _mid(num1, den1.reshape(_NP, 1), b1.reshape(1, _D), W2,
                  _acat(a_src2, a_dst2))
    num2, den2 = _edge_pass(h2, s2[:, 0], s2[:, 1], srcr, dstr)
    out = _fin(num2, den2.reshape(_NP, 1), b2.reshape(1, _D))
    return out[:_N]


# speedup vs baseline: 1.4861x; 1.0879x over previous
import functools

import jax
import jax.numpy as jnp
from jax import lax
from jax.experimental import pallas as pl
from jax.experimental.pallas import tpu as pltpu
from jax.experimental.pallas import tpu_sc as plsc

_N = 10000
_D = 128
_DG = 32
_NG = 4
_E = 320000
_ET = _E + _N
_NP = 10240
_NWS = 16
_C = 128
_NCH = 162
_NB = 6
_PF = 2
_EW = _NCH * _C
_EP = _NWS * _EW
_PAD_DST = 10200
_RPT = _NP // 16
_NEG_SLOPE = 0.2


_BLK = 512
_GRID = _NP // _BLK


def _proj_body(x_ref, w_ref, a_ref, h_ref, s_ref):
    h = jnp.dot(x_ref[...], w_ref[...], preferred_element_type=jnp.float32)
    for g in range(_NG):
        h_ref[g] = h[:, g * _DG:(g + 1) * _DG]
    s_ref[...] = jnp.dot(h, a_ref[...], preferred_element_type=jnp.float32)


_proj = pl.pallas_call(
    _proj_body,
    grid=(_GRID,),
    in_specs=[
        pl.BlockSpec((_BLK, _D), lambda i: (i, 0)),
        pl.BlockSpec((_D, _D), lambda i: (0, 0)),
        pl.BlockSpec((_D, _D), lambda i: (0, 0)),
    ],
    out_specs=[
        pl.BlockSpec((_NG, _BLK, _DG), lambda i: (0, i, 0)),
        pl.BlockSpec((_BLK, _D), lambda i: (i, 0)),
    ],
    out_shape=[
        jax.ShapeDtypeStruct((_NG, _NP, _DG), jnp.float32),
        jax.ShapeDtypeStruct((_NP, _D), jnp.float32),
    ],
)


def _mid_body(num_ref, den_ref, b_ref, w_ref, a_ref, h_ref, s_ref):
    num = jnp.concatenate([num_ref[g] for g in range(_NG)], axis=1)
    den = den_ref[...]
    out = num / (den + 1e-16) + b_ref[...]
    out = jnp.maximum(out, 0.0)
    h = jnp.dot(out, w_ref[...], preferred_element_type=jnp.float32)
    for g in range(_NG):
        h_ref[g] = h[:, g * _DG:(g + 1) * _DG]
    s_ref[...] = jnp.dot(h, a_ref[...], preferred_element_type=jnp.float32)


_mid = pl.pallas_call(
    _mid_body,
    grid=(_GRID,),
    in_specs=[
        pl.BlockSpec((_NG, _BLK, _DG), lambda i: (0, i, 0)),
        pl.BlockSpec((_BLK, 1), lambda i: (i, 0)),
        pl.BlockSpec((1, _D), lambda i: (0, 0)),
        pl.BlockSpec((_D, _D), lambda i: (0, 0)),
        pl.BlockSpec((_D, _D), lambda i: (0, 0)),
    ],
    out_specs=[
        pl.BlockSpec((_NG, _BLK, _DG), lambda i: (0, i, 0)),
        pl.BlockSpec((_BLK, _D), lambda i: (i, 0)),
    ],
    out_shape=[
        jax.ShapeDtypeStruct((_NG, _NP, _DG), jnp.float32),
        jax.ShapeDtypeStruct((_NP, _D), jnp.float32),
    ],
)


def _fin_body(num_ref, den_ref, b_ref, o_ref):
    num = jnp.concatenate([num_ref[g] for g in range(_NG)], axis=1)
    den = den_ref[...]
    out = num / (den + 1e-16) + b_ref[...]
    o_ref[...] = jnp.maximum(out, 0.0)


_fin = pl.pallas_call(
    _fin_body,
    grid=(_GRID,),
    in_specs=[
        pl.BlockSpec((_NG, _BLK, _DG), lambda i: (0, i, 0)),
        pl.BlockSpec((_BLK, 1), lambda i: (i, 0)),
        pl.BlockSpec((1, _D), lambda i: (0, 0)),
    ],
    out_specs=pl.BlockSpec((_BLK, _D), lambda i: (i, 0)),
    out_shape=jax.ShapeDtypeStruct((_NP, _D), jnp.float32),
)


_mesh = plsc.VectorSubcoreMesh(core_axis_name="c", subcore_axis_name="s")


@functools.partial(
    pl.kernel,
    mesh=_mesh,
    compiler_params=pltpu.CompilerParams(
        needs_layout_passes=False, use_tc_tiling_on_sc=False),
    out_type=[
        jax.ShapeDtypeStruct((_NG, _NP, _DG), jnp.float32),
        jax.ShapeDtypeStruct((1, _NP), jnp.float32),
    ],
    scratch_types=[
        pltpu.VMEM((_NCH, _C), jnp.int32),
        pltpu.VMEM((_NCH, _C), jnp.int32),
        pltpu.VMEM((_NP,), jnp.float32),
        pltpu.VMEM((_NP,), jnp.float32),
        pltpu.VMEM((_NCH, _C), jnp.float32),
    ] + [pltpu.VMEM((_C, _DG), jnp.float32) for _ in range(_NB)] + [
        pltpu.VMEM((640,), jnp.float32),
        pltpu.VMEM_SHARED((_NP, _DG), jnp.float32),
        pltpu.VMEM_SHARED((_NP,), jnp.float32),
    ] + [pltpu.SemaphoreType.DMA for _ in range(_NB + 2)],
)
def _edge_pass(h_hbm, as_hbm, ad_hbm, src_hbm, dst_hbm, num_out, den_out,
               src_v, dst_v, as_v, ad_v, t_v, *rest):
    rows = rest[:_NB]
    zden_v = rest[_NB]
    num_sh = rest[_NB + 1]
    den_sh = rest[_NB + 2]
    gsems = rest[_NB + 3:2 * _NB + 3]
    dsem = rest[2 * _NB + 3]
    ssem = rest[2 * _NB + 4]
    cid = lax.axis_index("c")
    sid = lax.axis_index("s")

    pltpu.sync_copy(src_hbm.at[sid], src_v)
    pltpu.sync_copy(dst_hbm.at[sid], dst_v)
    pltpu.sync_copy(as_hbm, as_v)
    pltpu.sync_copy(ad_hbm, ad_v)

    zeros16 = jnp.zeros((16,), jnp.float32)

    def _zden(i, _):
        zden_v[pl.ds(i * 16, 16)] = zeros16
        return 0

    lax.fori_loop(0, 640 // 16, _zden, 0)

    def _tloop(j, _):
        for g in range(8):
            sl = pl.ds(g * 16, 16)
            si = src_v[j, sl]
            di = dst_v[j, sl]
            e = plsc.load_gather(as_v, [si]) + plsc.load_gather(ad_v, [di])
            e = jnp.where(e > 0, e, _NEG_SLOPE * e)
            t_v[j, sl] = jnp.exp(e)
        return 0

    lax.fori_loop(0, _NCH, _tloop, 0)

    rbase = sid * _RPT

    for half in range(2):
        grp = cid * 2 + half

        def _zrows(i, _):
            r = i // (_DG // 16)
            g = i % (_DG // 16)
            rows[0][r, pl.ds(g * 16, 16)] = zeros16
            return 0

        lax.fori_loop(0, _C * (_DG // 16), _zrows, 0)
        for k in range(_RPT // _C):
            pltpu.sync_copy(rows[0], num_sh.at[pl.ds(rbase + k * _C, _C)])
        if _RPT % _C:
            pltpu.sync_copy(
                rows[0].at[pl.ds(0, _RPT % _C)],
                num_sh.at[pl.ds(rbase + (_RPT // _C) * _C, _RPT % _C)])
        if half == 0:
            pltpu.sync_copy(zden_v.at[pl.ds(0, _RPT)],
                            den_sh.at[pl.ds(rbase, _RPT)])

        plsc.subcore_barrier()

        h_grp = h_hbm.at[grp]

        for b in range(_NB):
            pltpu.async_copy(h_grp.at[src_v.at[b]], rows[b], gsems[b])

        def _outer(io, _):
            for k in range(_NB):
                j = io * _NB + k
                pltpu.make_async_copy(
                    h_grp.at[src_v.at[j]], rows[k], gsems[k]).wait()
                if half == 0:
                    dcopy = pltpu.async_copy(
                        t_v.at[j], den_sh.at[dst_v.at[j]], dsem, add=True)

                def _scale(q, _c, k=k, j=j):
                    t16 = t_v[j, pl.ds(q * 16, 16)]
                    for l in range(16):
                        ts = t16[l]
                        r = q * 16 + l
                        for g in range(_DG // 16):
                            sl = pl.ds(g * 16, 16)
                            rows[k][r, sl] = rows[k][r, sl] * ts
                    return 0

                lax.fori_loop(0, _C // 16, _scale, 0)
                pltpu.async_copy(
                    rows[k], num_sh.at[dst_v.at[j]], ssem, add=True)
                if half == 0:
                    dcopy.wait()

                kp = (k - 2) % _NB
                jp = j - 2

                @pl.when(jp >= 0)
                def _(kp=kp, jp=jp):
                    pltpu.make_async_copy(
                        rows[kp], num_sh.at[dst_v.at[jp]], ssem).wait()

                @pl.when((jp >= 0) & (j + _NB - 2 < _NCH))
                def _(kp=kp, j=j):
                    pltpu.async_copy(
                        h_grp.at[src_v.at[j + _NB - 2]], rows[kp], gsems[kp])
            return 0

        lax.fori_loop(0, _NCH // _NB, _outer, 0)

        for i in range(2):
            j = _NCH - 2 + i
            k = j % _NB
            pltpu.make_async_copy(
                rows[k], num_sh.at[dst_v.at[j]], ssem).wait()

        plsc.subcore_barrier()

        pltpu.sync_copy(num_sh.at[pl.ds(rbase, _RPT)],
                        num_out.at[grp, pl.ds(rbase, _RPT)])
        if half == 0:
            @pl.when(cid == 0)
            def _():
                pltpu.sync_copy(den_sh.at[pl.ds(rbase, _RPT)],
                                den_out.at[0, pl.ds(rbase, _RPT)])

        plsc.subcore_barrier()


def _acat(a_s, a_d):
    z = jnp.zeros((_D, _D - 2), jnp.float32)
    return jnp.concatenate([a_s[:, None], a_d[:, None], z], axis=1)


def kernel(x, edge_index, W1, a_src1, a_dst1, b1, W2, a_src2, a_dst2, b2):
    xp = jnp.pad(x, ((0, _NP - _N), (0, 0)))

    ar = jnp.arange(_N, dtype=edge_index.dtype)
    src = jnp.concatenate([edge_index[0], ar])
    dst = jnp.concatenate([edge_index[1], ar])
    src = jnp.pad(src, (0, _EP - _ET))
    dst = jnp.pad(dst, (0, _EP - _ET), constant_values=_PAD_DST)
    srcr = src.reshape(_NWS, _NCH, _C)
    dstr = dst.reshape(_NWS, _NCH, _C)

    h1, s1 = _proj(xp, W1, _acat(a_src1, a_dst1))
    num1, den1 = _edge_pass(h1, s1[:, 0], s1[:, 1], srcr, dstr)
    h2, s2 = _mid(num1, den1.reshape(_NP, 1), b1.reshape(1, _D), W2,
                  _acat(a_src2, a_dst2))
    num2, den2 = _edge_pass(h2, s2[:, 0], s2[:, 1], srcr, dstr)
    out = _fin(num2, den2.reshape(_NP, 1), b2.reshape(1, _D))
    return out[:_N]


# speedup vs baseline: 1.4976x; 1.0077x over previous
import functools

import jax
import jax.numpy as jnp
from jax import lax
from jax.experimental import pallas as pl
from jax.experimental.pallas import tpu as pltpu
from jax.experimental.pallas import tpu_sc as plsc

_N = 10000
_D = 128
_DG = 32
_NG = 4
_E = 320000
_ET = _E + _N
_NP = 10240
_NWS = 16
_C = 128
_NCH = 162
_NB = 6
_PF = 2
_EW = _NCH * _C
_EP = _NWS * _EW
_PAD_DST = 10200
_RPT = _NP // 16
_NEG_SLOPE = 0.2


_BLK = 512
_GRID = _NP // _BLK


def _proj_body(x_ref, w_ref, a_ref, h_ref, s_ref):
    h = jnp.dot(x_ref[...], w_ref[...], preferred_element_type=jnp.float32)
    for g in range(_NG):
        h_ref[g] = h[:, g * _DG:(g + 1) * _DG]
    s_ref[...] = jnp.dot(h, a_ref[...], preferred_element_type=jnp.float32)


_proj = pl.pallas_call(
    _proj_body,
    grid=(_GRID,),
    in_specs=[
        pl.BlockSpec((_BLK, _D), lambda i: (i, 0)),
        pl.BlockSpec((_D, _D), lambda i: (0, 0)),
        pl.BlockSpec((_D, _D), lambda i: (0, 0)),
    ],
    out_specs=[
        pl.BlockSpec((_NG, _BLK, _DG), lambda i: (0, i, 0)),
        pl.BlockSpec((_BLK, _D), lambda i: (i, 0)),
    ],
    out_shape=[
        jax.ShapeDtypeStruct((_NG, _NP, _DG), jnp.float32),
        jax.ShapeDtypeStruct((_NP, _D), jnp.float32),
    ],
)


def _mid_body(num_ref, den_ref, b_ref, w_ref, a_ref, h_ref, s_ref):
    num = jnp.concatenate([num_ref[g] for g in range(_NG)], axis=1)
    den = den_ref[...]
    out = num / (den + 1e-16) + b_ref[...]
    out = jnp.maximum(out, 0.0)
    h = jnp.dot(out, w_ref[...], preferred_element_type=jnp.float32)
    for g in range(_NG):
        h_ref[g] = h[:, g * _DG:(g + 1) * _DG]
    s_ref[...] = jnp.dot(h, a_ref[...], preferred_element_type=jnp.float32)


_mid = pl.pallas_call(
    _mid_body,
    grid=(_GRID,),
    in_specs=[
        pl.BlockSpec((_NG, _BLK, _DG), lambda i: (0, i, 0)),
        pl.BlockSpec((_BLK, 1), lambda i: (i, 0)),
        pl.BlockSpec((1, _D), lambda i: (0, 0)),
        pl.BlockSpec((_D, _D), lambda i: (0, 0)),
        pl.BlockSpec((_D, _D), lambda i: (0, 0)),
    ],
    out_specs=[
        pl.BlockSpec((_NG, _BLK, _DG), lambda i: (0, i, 0)),
        pl.BlockSpec((_BLK, _D), lambda i: (i, 0)),
    ],
    out_shape=[
        jax.ShapeDtypeStruct((_NG, _NP, _DG), jnp.float32),
        jax.ShapeDtypeStruct((_NP, _D), jnp.float32),
    ],
)


def _fin_body(num_ref, den_ref, b_ref, o_ref):
    num = jnp.concatenate([num_ref[g] for g in range(_NG)], axis=1)
    den = den_ref[...]
    out = num / (den + 1e-16) + b_ref[...]
    o_ref[...] = jnp.maximum(out, 0.0)


_fin = pl.pallas_call(
    _fin_body,
    grid=(_GRID,),
    in_specs=[
        pl.BlockSpec((_NG, _BLK, _DG), lambda i: (0, i, 0)),
        pl.BlockSpec((_BLK, 1), lambda i: (i, 0)),
        pl.BlockSpec((1, _D), lambda i: (0, 0)),
    ],
    out_specs=pl.BlockSpec((_BLK, _D), lambda i: (i, 0)),
    out_shape=jax.ShapeDtypeStruct((_NP, _D), jnp.float32),
)


_mesh = plsc.VectorSubcoreMesh(core_axis_name="c", subcore_axis_name="s")


@functools.partial(
    pl.kernel,
    mesh=_mesh,
    compiler_params=pltpu.CompilerParams(
        needs_layout_passes=False, use_tc_tiling_on_sc=False),
    out_type=[
        jax.ShapeDtypeStruct((_NG, _NP, _DG), jnp.float32),
        jax.ShapeDtypeStruct((1, _NP), jnp.float32),
    ],
    scratch_types=[
        pltpu.VMEM((_NCH, _C), jnp.int32),
        pltpu.VMEM((_NCH, _C), jnp.int32),
        pltpu.VMEM((_NP,), jnp.float32),
        pltpu.VMEM((_NP,), jnp.float32),
        pltpu.VMEM((_NCH, _C), jnp.float32),
    ] + [pltpu.VMEM((_C, _DG), jnp.float32) for _ in range(_NB)] + [
        pltpu.VMEM((640,), jnp.float32),
        pltpu.VMEM_SHARED((_NP, _DG), jnp.float32),
        pltpu.VMEM_SHARED((_NP,), jnp.float32),
    ] + [pltpu.SemaphoreType.DMA for _ in range(_NB + 2)],
)
def _edge_pass(h_hbm, as_hbm, ad_hbm, src_hbm, dst_hbm, num_out, den_out,
               src_v, dst_v, as_v, ad_v, t_v, *rest):
    rows = rest[:_NB]
    zden_v = rest[_NB]
    num_sh = rest[_NB + 1]
    den_sh = rest[_NB + 2]
    gsems = rest[_NB + 3:2 * _NB + 3]
    dsem = rest[2 * _NB + 3]
    ssem = rest[2 * _NB + 4]
    cid = lax.axis_index("c")
    sid = lax.axis_index("s")

    pltpu.sync_copy(src_hbm.at[sid], src_v)
    pltpu.sync_copy(dst_hbm.at[sid], dst_v)
    pltpu.sync_copy(as_hbm, as_v)
    pltpu.sync_copy(ad_hbm, ad_v)

    zeros16 = jnp.zeros((16,), jnp.float32)

    def _zden(i, _):
        zden_v[pl.ds(i * 16, 16)] = zeros16
        return 0

    lax.fori_loop(0, 640 // 16, _zden, 0)

    def _tloop(j, _):
        for g in range(8):
            sl = pl.ds(g * 16, 16)
            si = src_v[j, sl]
            di = dst_v[j, sl]
            e = plsc.load_gather(as_v, [si]) + plsc.load_gather(ad_v, [di])
            e = jnp.where(e > 0, e, _NEG_SLOPE * e)
            t_v[j, sl] = jnp.exp(e)
        return 0

    lax.fori_loop(0, _NCH, _tloop, 0)

    rbase = sid * _RPT

    for half in range(2):
        grp = cid * 2 + half

        def _zrows(i, _):
            r = i // (_DG // 16)
            g = i % (_DG // 16)
            rows[0][r, pl.ds(g * 16, 16)] = zeros16
            return 0

        lax.fori_loop(0, _C * (_DG // 16), _zrows, 0)
        for k in range(_RPT // _C):
            pltpu.sync_copy(rows[0], num_sh.at[pl.ds(rbase + k * _C, _C)])
        if _RPT % _C:
            pltpu.sync_copy(
                rows[0].at[pl.ds(0, _RPT % _C)],
                num_sh.at[pl.ds(rbase + (_RPT // _C) * _C, _RPT % _C)])
        if half == 0:
            pltpu.sync_copy(zden_v.at[pl.ds(0, _RPT)],
                            den_sh.at[pl.ds(rbase, _RPT)])

        plsc.subcore_barrier()

        h_grp = h_hbm.at[grp]

        for b in range(_NB):
            pltpu.async_copy(h_grp.at[src_v.at[b]], rows[b], gsems[b])

        def _outer(io, _):
            for k in range(_NB):
                j = io * _NB + k
                pltpu.make_async_copy(
                    h_grp.at[src_v.at[j]], rows[k], gsems[k]).wait()
                if half == 0:
                    dcopy = pltpu.async_copy(
                        t_v.at[j], den_sh.at[dst_v.at[j]], dsem, add=True)

                @plsc.parallel_loop(0, _C // 16, unroll=2)
                def _scale(q, k=k, j=j):
                    t16 = t_v[j, pl.ds(q * 16, 16)]
                    for l in range(16):
                        ts = t16[l]
                        r = q * 16 + l
                        for g in range(_DG // 16):
                            sl = pl.ds(g * 16, 16)
                            rows[k][r, sl] = rows[k][r, sl] * ts
                pltpu.async_copy(
                    rows[k], num_sh.at[dst_v.at[j]], ssem, add=True)
                if half == 0:
                    dcopy.wait()

                kp = (k - 2) % _NB
                jp = j - 2

                @pl.when(jp >= 0)
                def _(kp=kp, jp=jp):
                    pltpu.make_async_copy(
                        rows[kp], num_sh.at[dst_v.at[jp]], ssem).wait()

                @pl.when((jp >= 0) & (j + _NB - 2 < _NCH))
                def _(kp=kp, j=j):
                    pltpu.async_copy(
                        h_grp.at[src_v.at[j + _NB - 2]], rows[kp], gsems[kp])
            return 0

        lax.fori_loop(0, _NCH // _NB, _outer, 0)

        for i in range(2):
            j = _NCH - 2 + i
            k = j % _NB
            pltpu.make_async_copy(
                rows[k], num_sh.at[dst_v.at[j]], ssem).wait()

        plsc.subcore_barrier()

        pltpu.sync_copy(num_sh.at[pl.ds(rbase, _RPT)],
                        num_out.at[grp, pl.ds(rbase, _RPT)])
        if half == 0:
            @pl.when(cid == 0)
            def _():
                pltpu.sync_copy(den_sh.at[pl.ds(rbase, _RPT)],
                                den_out.at[0, pl.ds(rbase, _RPT)])

        plsc.subcore_barrier()


def _acat(a_s, a_d):
    z = jnp.zeros((_D, _D - 2), jnp.float32)
    return jnp.concatenate([a_s[:, None], a_d[:, None], z], axis=1)


def kernel(x, edge_index, W1, a_src1, a_dst1, b1, W2, a_src2, a_dst2, b2):
    xp = jnp.pad(x, ((0, _NP - _N), (0, 0)))

    ar = jnp.arange(_N, dtype=edge_index.dtype)
    src = jnp.concatenate([edge_index[0], ar])
    dst = jnp.concatenate([edge_index[1], ar])
    src = jnp.pad(src, (0, _EP - _ET))
    dst = jnp.pad(dst, (0, _EP - _ET), constant_values=_PAD_DST)
    srcr = src.reshape(_NWS, _NCH, _C)
    dstr = dst.reshape(_NWS, _NCH, _C)

    h1, s1 = _proj(xp, W1, _acat(a_src1, a_dst1))
    num1, den1 = _edge_pass(h1, s1[:, 0], s1[:, 1], srcr, dstr)
    h2, s2 = _mid(num1, den1.reshape(_NP, 1), b1.reshape(1, _D), W2,
                  _acat(a_src2, a_dst2))
    num2, den2 = _edge_pass(h2, s2[:, 0], s2[:, 1], srcr, dstr)
    out = _fin(num2, den2.reshape(_NP, 1), b2.reshape(1, _D))
    return out[:_N]
